# baseline (speedup 1.0000x reference)
import functools

import jax
import jax.numpy as jnp
from jax import lax
from jax.experimental import pallas as pl
from jax.experimental.pallas import tpu as pltpu
from jax.experimental.pallas import tpu_sc as plsc

_RADIUS = 6371.0
_KNN = 10
_M = 16
_NWORKER = 32
_CHUNK = 20000
_LANES = 16

_SIN_C = [1.0, -1.0 / 6, 1.0 / 120, -1.0 / 5040, 1.0 / 362880,
          -1.0 / 39916800]
_COS_C = [1.0, -0.5, 1.0 / 24, -1.0 / 720, 1.0 / 40320, -1.0 / 3628800,
          1.0 / 479001600]
_ASIN_C = []
_c = 1.0
for _n in range(14):
    if _n:
        _c *= ((2 * _n - 1) / (2 * _n)) * ((2 * _n - 1) / (2 * _n + 1))
    _ASIN_C.append(_c)
del _c, _n


def _poly_u(u, coeffs):
    p = jnp.float32(coeffs[-1])
    for c in reversed(coeffs[:-1]):
        p = p * u + jnp.float32(c)
    return p


def _sinp(t):
    return t * _poly_u(t * t, _SIN_C)


def _cosp(t):
    return _poly_u(t * t, _COS_C)


def _asinp(x):
    return x * _poly_u(x * x, _ASIN_C)


def _feat_body(lat_ref, lon_ref, x_ref, y_ref, z_ref):
    lat = lat_ref[...]
    lon = lon_ref[...]
    cl = _cosp(lat)
    x_ref[...] = _sinp(lat)
    y_ref[...] = cl * _cosp(lon)
    z_ref[...] = cl * _sinp(lon)


def _features(lat2d, lon2d):
    out = jax.ShapeDtypeStruct(lat2d.shape, jnp.float32)
    return pl.pallas_call(_feat_body, out_shape=[out, out, out])(lat2d, lon2d)


def _sc_body(nchunk, qpw,
             qx_h, qy_h, qz_h, ox_h, oy_h, oz_h, olat_h, olon_h, otgt_h,
             oi_h, clat_h, clon_h, ctgt_h,
             qxv, qyv, qzv, oxv, oyv, ozv, bdv, biv, g0, g1, g2, sem):
    wid = lax.axis_index("s") * 2 + lax.axis_index("c")
    qb = wid * qpw
    ncand = qpw * _M

    pltpu.sync_copy(qx_h.at[pl.ds(qb * _LANES, qpw * _LANES)], qxv)
    pltpu.sync_copy(qy_h.at[pl.ds(qb * _LANES, qpw * _LANES)], qyv)
    pltpu.sync_copy(qz_h.at[pl.ds(qb * _LANES, qpw * _LANES)], qzv)

    def init_q(q, carry):
        bdv[pl.ds(q * _M, _M)] = jnp.full((_LANES,), jnp.inf, jnp.float32)
        biv[q >> 3, pl.ds((q & 7) * _M, _M)] = jnp.zeros((_LANES,), jnp.int32)
        return carry

    lax.fori_loop(0, qpw, init_q, 0)

    for ci in range(nchunk):
        pltpu.sync_copy(ox_h.at[pl.ds(ci * _CHUNK, _CHUNK)], oxv)
        pltpu.sync_copy(oy_h.at[pl.ds(ci * _CHUNK, _CHUNK)], oyv)
        pltpu.sync_copy(oz_h.at[pl.ds(ci * _CHUNK, _CHUNK)], ozv)

        def q_loop(q, carry):
            qx = qxv[pl.ds(q * _LANES, _LANES)]
            qy = qyv[pl.ds(q * _LANES, _LANES)]
            qz = qzv[pl.ds(q * _LANES, _LANES)]
            bd0 = bdv[pl.ds(q * _M, _M)]
            bi0 = biv[q >> 3, pl.ds((q & 7) * _M, _M)]
            thr0 = jnp.full((_LANES,), bd0[_M - 1], jnp.float32)

            def v_body(i, car):
                bd, bi, thr = car
                off = i * _LANES
                dx = oxv[pl.ds(off, _LANES)] - qx
                dy = oyv[pl.ds(off, _LANES)] - qy
                dz = ozv[pl.ds(off, _LANES)] - qz
                d2 = dx * dx + dy * dy + dz * dz
                m = d2 < thr

                def ins():
                    idxv = (jnp.int32(ci * _CHUNK) + off
                            + lax.iota(jnp.int32, _LANES))
                    dm = jnp.where(m, d2, jnp.inf)
                    sd, si = plsc.sort_key_val(dm, idxv, descending=True)
                    lo = jnp.minimum(bd, sd)
                    ni = jnp.where(bd <= sd, bi, si)
                    nd, nid = plsc.sort_key_val(lo, ni)
                    return nd, nid, jnp.full((_LANES,), nd[_M - 1],
                                             jnp.float32)

                nhit = plsc.all_reduce_population_count(m)
                return lax.cond(nhit[0] > 0, ins, lambda: (bd, bi, thr))

            bdq, biq, _ = lax.fori_loop(0, _CHUNK // _LANES, v_body,
                                        (bd0, bi0, thr0))
            bdv[pl.ds(q * _M, _M)] = bdq
            biv[q >> 3, pl.ds((q & 7) * _M, _M)] = biq
            return carry

        lax.fori_loop(0, qpw, q_loop, 0)

    for p in range(ncand // 128):
        pltpu.async_copy(olat_h.at[biv.at[p]],
                         g0.at[pl.ds(p * 128, 128)], sem).wait()
        pltpu.async_copy(olon_h.at[biv.at[p]],
                         g1.at[pl.ds(p * 128, 128)], sem).wait()
        pltpu.async_copy(otgt_h.at[biv.at[p]],
                         g2.at[pl.ds(p * 128, 128)], sem).wait()

    for p in range(ncand // 128):
        pltpu.sync_copy(biv.at[p], oi_h.at[pl.ds(qb * _M + p * 128, 128)])
    pltpu.sync_copy(g0, clat_h.at[pl.ds(qb * _M, ncand)])
    pltpu.sync_copy(g1, clon_h.at[pl.ds(qb * _M, ncand)])
    pltpu.sync_copy(g2, ctgt_h.at[pl.ds(qb * _M, ncand)])


def _topk_sc(qx, qy, qz, ox, oy, oz, obs_lat, obs_lon, obs_tgt):
    q = qx.shape[0] // _LANES
    k = ox.shape[0]
    assert k % _CHUNK == 0 and q % _NWORKER == 0
    qpw = q // _NWORKER
    ncand = qpw * _M
    mesh = plsc.VectorSubcoreMesh(core_axis_name="c", subcore_axis_name="s")
    f32, i32 = jnp.float32, jnp.int32
    fn = functools.partial(_sc_body, k // _CHUNK, qpw)
    kfn = pl.kernel(
        fn,
        mesh=mesh,
        compiler_params=pltpu.CompilerParams(needs_layout_passes=False),
        out_type=[
            jax.ShapeDtypeStruct((q * _M,), i32),
            jax.ShapeDtypeStruct((q * _M,), f32),
            jax.ShapeDtypeStruct((q * _M,), f32),
            jax.ShapeDtypeStruct((q * _M,), f32),
        ],
        scratch_types=[
            pltpu.VMEM((qpw * _LANES,), f32),
            pltpu.VMEM((qpw * _LANES,), f32),
            pltpu.VMEM((qpw * _LANES,), f32),
            pltpu.VMEM((_CHUNK,), f32),
            pltpu.VMEM((_CHUNK,), f32),
            pltpu.VMEM((_CHUNK,), f32),
            pltpu.VMEM((ncand,), f32),
            pltpu.VMEM((ncand // 128, 128), i32),
            pltpu.VMEM((ncand,), f32),
            pltpu.VMEM((ncand,), f32),
            pltpu.VMEM((ncand,), f32),
            pltpu.SemaphoreType.DMA,
        ],
    )
    return kfn(qx, qy, qz, ox, oy, oz, obs_lat, obs_lon, obs_tgt)


def _refine_body(qlat_ref, qlon_ref, klat_ref, klon_ref, ktgt_ref, kidx_ref,
                 d_ref, i_ref, t_ref):
    ql = qlat_ref[...]
    qn = qlon_ref[...]
    kl = klat_ref[...]
    kn = klon_ref[...]
    kt = ktgt_ref[...]
    ki = kidx_ref[...]

    sdlat = _sinp((ql - kl) * 0.5)
    sdlon = _sinp((qn - kn) * 0.5)
    a = sdlat * sdlat + _cosp(ql) * _cosp(kl) * sdlon * sdlon
    a = jnp.clip(a, 1e-12, 1.0)
    d_all = (2.0 * _RADIUS) * _asinp(jnp.sqrt(a))

    big = jnp.int32(2 ** 30)
    col = lax.broadcasted_iota(jnp.int32, a.shape, 1)
    vals = a
    out_d = jnp.zeros(a.shape, jnp.float32)
    out_i = jnp.zeros(a.shape, jnp.int32)
    out_t = jnp.zeros(a.shape, jnp.float32)
    for j in range(_KNN):
        mn = jnp.min(vals, axis=1, keepdims=True)
        ismn = vals == mn
        pstar = jnp.min(jnp.where(ismn, ki, big), axis=1, keepdims=True)
        pick = ismn & (ki == pstar)
        dj = jnp.sum(jnp.where(pick, d_all, 0.0), axis=1, keepdims=True)
        tj = jnp.sum(jnp.where(pick, kt, 0.0), axis=1, keepdims=True)
        out_d = jnp.where(col == j, dj, out_d)
        out_i = jnp.where(col == j, pstar, out_i)
        out_t = jnp.where(col == j, tj, out_t)
        vals = jnp.where(pick, jnp.inf, vals)

    d_ref[...] = out_d[:, :_KNN]
    i_ref[...] = out_i[:, :_KNN]
    t_ref[...] = out_t[:, :_KNN]


def _refine(qlat, qlon, clat, clon, ctgt, cidx):
    q = qlat.shape[0]
    return pl.pallas_call(
        _refine_body,
        out_shape=[
            jax.ShapeDtypeStruct((q, _KNN), jnp.float32),
            jax.ShapeDtypeStruct((q, _KNN), jnp.int32),
            jax.ShapeDtypeStruct((q, _KNN), jnp.float32),
        ],
    )(qlat, qlon, clat, clon, ctgt, cidx)


def kernel(query_lat, query_lon, obs_lat, obs_lon, obs_targets):
    q = query_lat.shape[0]
    k = obs_lat.shape[0]
    kp = ((k + 127) // 128) * 128

    olat2 = jnp.pad(obs_lat, (0, kp - k)).reshape(kp // 128, 128)
    olon2 = jnp.pad(obs_lon, (0, kp - k)).reshape(kp // 128, 128)
    ox2, oy2, oz2 = _features(olat2, olon2)
    ox = ox2.reshape(kp)[:k]
    oy = oy2.reshape(kp)[:k]
    oz = oz2.reshape(kp)[:k]

    qx2, qy2, qz2 = _features(query_lat.reshape(q // 128, 128),
                              query_lon.reshape(q // 128, 128))
    qx = qx2.reshape(q)
    qy = qy2.reshape(q)
    qz = qz2.reshape(q)

    oi, clat, clon, ctgt = _topk_sc(
        jnp.repeat(qx, _LANES), jnp.repeat(qy, _LANES),
        jnp.repeat(qz, _LANES), ox, oy, oz,
        obs_lat, obs_lon, obs_targets)

    dists, indices, targets = _refine(
        query_lat.reshape(q, 1), query_lon.reshape(q, 1),
        clat.reshape(q, _M), clon.reshape(q, _M), ctgt.reshape(q, _M),
        oi.reshape(q, _M))
    return dists, indices, targets


# speedup vs baseline: 3.7415x; 3.7415x over previous
import functools

import jax
import jax.numpy as jnp
from jax import lax
from jax.experimental import pallas as pl
from jax.experimental.pallas import tpu as pltpu
from jax.experimental.pallas import tpu_sc as plsc

_RADIUS = 6371.0
_KNN = 10
_M = 16
_NWORKER = 32
_CHUNK = 20000
_LANES = 16
_G = 5

_SIN_C = [1.0, -1.0 / 6, 1.0 / 120, -1.0 / 5040, 1.0 / 362880,
          -1.0 / 39916800]
_COS_C = [1.0, -0.5, 1.0 / 24, -1.0 / 720, 1.0 / 40320, -1.0 / 3628800,
          1.0 / 479001600]
_ASIN_C = []
_c = 1.0
for _n in range(14):
    if _n:
        _c *= ((2 * _n - 1) / (2 * _n)) * ((2 * _n - 1) / (2 * _n + 1))
    _ASIN_C.append(_c)
del _c, _n


def _poly_u(u, coeffs):
    p = jnp.float32(coeffs[-1])
    for c in reversed(coeffs[:-1]):
        p = p * u + jnp.float32(c)
    return p


def _sinp(t):
    return t * _poly_u(t * t, _SIN_C)


def _cosp(t):
    return _poly_u(t * t, _COS_C)


def _asinp(x):
    return x * _poly_u(x * x, _ASIN_C)


def _feat_body(lat_ref, lon_ref, x_ref, y_ref, z_ref):
    lat = lat_ref[...]
    lon = lon_ref[...]
    cl = _cosp(lat)
    x_ref[...] = _sinp(lat)
    y_ref[...] = cl * _cosp(lon)
    z_ref[...] = cl * _sinp(lon)


def _features(lat2d, lon2d):
    out = jax.ShapeDtypeStruct(lat2d.shape, jnp.float32)
    return pl.pallas_call(_feat_body, out_shape=[out, out, out])(lat2d, lon2d)


def _sc_body(nchunk, qpw,
             qx_h, qy_h, qz_h, ox_h, oy_h, oz_h, olat_h, olon_h, otgt_h,
             oi_h, clat_h, clon_h, ctgt_h,
             qxv, qyv, qzv, oxv, oyv, ozv, bdv, biv, g0, g1, g2, sem):
    wid = lax.axis_index("s") * 2 + lax.axis_index("c")
    qb = wid * qpw
    ncand = qpw * _M

    pltpu.sync_copy(qx_h.at[pl.ds(qb * _LANES, qpw * _LANES)], qxv)
    pltpu.sync_copy(qy_h.at[pl.ds(qb * _LANES, qpw * _LANES)], qyv)
    pltpu.sync_copy(qz_h.at[pl.ds(qb * _LANES, qpw * _LANES)], qzv)

    def init_q(q, carry):
        bdv[pl.ds(q * _M, _M)] = jnp.full((_LANES,), jnp.inf, jnp.float32)
        biv[q >> 3, pl.ds((q & 7) * _M, _M)] = jnp.zeros((_LANES,), jnp.int32)
        return carry

    lax.fori_loop(0, qpw, init_q, 0)

    for ci in range(nchunk):
        pltpu.sync_copy(ox_h.at[pl.ds(ci * _CHUNK, _CHUNK)], oxv)
        pltpu.sync_copy(oy_h.at[pl.ds(ci * _CHUNK, _CHUNK)], oyv)
        pltpu.sync_copy(oz_h.at[pl.ds(ci * _CHUNK, _CHUNK)], ozv)

        def q_loop(q, carry):
            qx = qxv[pl.ds(q * _LANES, _LANES)]
            qy = qyv[pl.ds(q * _LANES, _LANES)]
            qz = qzv[pl.ds(q * _LANES, _LANES)]
            bd0 = bdv[pl.ds(q * _M, _M)]
            bi0 = biv[q >> 3, pl.ds((q & 7) * _M, _M)]
            thr0 = jnp.full((_LANES,), bd0[_M - 1], jnp.float32)

            def g_body(g, car):
                bd, bi, thr = car
                off0 = g * (_LANES * _G)
                d2s = []
                for t in range(_G):
                    o = off0 + t * _LANES
                    dx = oxv[pl.ds(o, _LANES)] - qx
                    dy = oyv[pl.ds(o, _LANES)] - qy
                    dz = ozv[pl.ds(o, _LANES)] - qz
                    d2s.append(dx * dx + dy * dy + dz * dz)
                mn = d2s[0]
                for t in range(1, _G):
                    mn = jnp.minimum(mn, d2s[t])

                def ins_group():
                    car2 = (bd, bi, thr)
                    for t in range(_G):
                        bd_c, bi_c, thr_c = car2
                        d2 = d2s[t]
                        m = d2 < thr_c

                        def ins_t(d2=d2, m=m, bd_c=bd_c, bi_c=bi_c, t=t):
                            idxv = (jnp.int32(ci * _CHUNK) + off0
                                    + jnp.int32(t * _LANES)
                                    + lax.iota(jnp.int32, _LANES))
                            dm = jnp.where(m, d2, jnp.inf)
                            sd, si = plsc.sort_key_val(dm, idxv,
                                                       descending=True)
                            lo = jnp.minimum(bd_c, sd)
                            ni = jnp.where(bd_c <= sd, bi_c, si)
                            nd, nid = plsc.sort_key_val(lo, ni)
                            return nd, nid, jnp.full((_LANES,), nd[_M - 1],
                                                     jnp.float32)

                        nh = plsc.all_reduce_population_count(m)
                        car2 = lax.cond(
                            nh[0] > 0, ins_t,
                            lambda a=bd_c, b=bi_c, c=thr_c: (a, b, c))
                    return car2

                anyhit = plsc.all_reduce_population_count(mn < thr)
                return lax.cond(anyhit[0] > 0, ins_group,
                                lambda: (bd, bi, thr))

            bdq, biq, _ = lax.fori_loop(0, _CHUNK // (_LANES * _G), g_body,
                                        (bd0, bi0, thr0))
            bdv[pl.ds(q * _M, _M)] = bdq
            biv[q >> 3, pl.ds((q & 7) * _M, _M)] = biq
            return carry

        lax.fori_loop(0, qpw, q_loop, 0)

    for p in range(ncand // 128):
        pltpu.async_copy(olat_h.at[biv.at[p]],
                         g0.at[pl.ds(p * 128, 128)], sem).wait()
        pltpu.async_copy(olon_h.at[biv.at[p]],
                         g1.at[pl.ds(p * 128, 128)], sem).wait()
        pltpu.async_copy(otgt_h.at[biv.at[p]],
                         g2.at[pl.ds(p * 128, 128)], sem).wait()

    for p in range(ncand // 128):
        pltpu.sync_copy(biv.at[p], oi_h.at[pl.ds(qb * _M + p * 128, 128)])
    pltpu.sync_copy(g0, clat_h.at[pl.ds(qb * _M, ncand)])
    pltpu.sync_copy(g1, clon_h.at[pl.ds(qb * _M, ncand)])
    pltpu.sync_copy(g2, ctgt_h.at[pl.ds(qb * _M, ncand)])


def _topk_sc(qx, qy, qz, ox, oy, oz, obs_lat, obs_lon, obs_tgt):
    q = qx.shape[0] // _LANES
    k = ox.shape[0]
    assert k % _CHUNK == 0 and q % _NWORKER == 0
    qpw = q // _NWORKER
    ncand = qpw * _M
    mesh = plsc.VectorSubcoreMesh(core_axis_name="c", subcore_axis_name="s")
    f32, i32 = jnp.float32, jnp.int32
    fn = functools.partial(_sc_body, k // _CHUNK, qpw)
    kfn = pl.kernel(
        fn,
        mesh=mesh,
        compiler_params=pltpu.CompilerParams(needs_layout_passes=False),
        out_type=[
            jax.ShapeDtypeStruct((q * _M,), i32),
            jax.ShapeDtypeStruct((q * _M,), f32),
            jax.ShapeDtypeStruct((q * _M,), f32),
            jax.ShapeDtypeStruct((q * _M,), f32),
        ],
        scratch_types=[
            pltpu.VMEM((qpw * _LANES,), f32),
            pltpu.VMEM((qpw * _LANES,), f32),
            pltpu.VMEM((qpw * _LANES,), f32),
            pltpu.VMEM((_CHUNK,), f32),
            pltpu.VMEM((_CHUNK,), f32),
            pltpu.VMEM((_CHUNK,), f32),
            pltpu.VMEM((ncand,), f32),
            pltpu.VMEM((ncand // 128, 128), i32),
            pltpu.VMEM((ncand,), f32),
            pltpu.VMEM((ncand,), f32),
            pltpu.VMEM((ncand,), f32),
            pltpu.SemaphoreType.DMA,
        ],
    )
    return kfn(qx, qy, qz, ox, oy, oz, obs_lat, obs_lon, obs_tgt)


def _refine_body(qlat_ref, qlon_ref, klat_ref, klon_ref, ktgt_ref, kidx_ref,
                 d_ref, i_ref, t_ref):
    ql = qlat_ref[...]
    qn = qlon_ref[...]
    kl = klat_ref[...]
    kn = klon_ref[...]
    kt = ktgt_ref[...]
    ki = kidx_ref[...]

    sdlat = _sinp((ql - kl) * 0.5)
    sdlon = _sinp((qn - kn) * 0.5)
    a = sdlat * sdlat + _cosp(ql) * _cosp(kl) * sdlon * sdlon
    a = jnp.clip(a, 1e-12, 1.0)
    d_all = (2.0 * _RADIUS) * _asinp(jnp.sqrt(a))

    big = jnp.int32(2 ** 30)
    col = lax.broadcasted_iota(jnp.int32, a.shape, 1)
    vals = a
    out_d = jnp.zeros(a.shape, jnp.float32)
    out_i = jnp.zeros(a.shape, jnp.int32)
    out_t = jnp.zeros(a.shape, jnp.float32)
    for j in range(_KNN):
        mn = jnp.min(vals, axis=1, keepdims=True)
        ismn = vals == mn
        pstar = jnp.min(jnp.where(ismn, ki, big), axis=1, keepdims=True)
        pick = ismn & (ki == pstar)
        dj = jnp.sum(jnp.where(pick, d_all, 0.0), axis=1, keepdims=True)
        tj = jnp.sum(jnp.where(pick, kt, 0.0), axis=1, keepdims=True)
        out_d = jnp.where(col == j, dj, out_d)
        out_i = jnp.where(col == j, pstar, out_i)
        out_t = jnp.where(col == j, tj, out_t)
        vals = jnp.where(pick, jnp.inf, vals)

    d_ref[...] = out_d[:, :_KNN]
    i_ref[...] = out_i[:, :_KNN]
    t_ref[...] = out_t[:, :_KNN]


def _refine(qlat, qlon, clat, clon, ctgt, cidx):
    q = qlat.shape[0]
    return pl.pallas_call(
        _refine_body,
        out_shape=[
            jax.ShapeDtypeStruct((q, _KNN), jnp.float32),
            jax.ShapeDtypeStruct((q, _KNN), jnp.int32),
            jax.ShapeDtypeStruct((q, _KNN), jnp.float32),
        ],
    )(qlat, qlon, clat, clon, ctgt, cidx)


def kernel(query_lat, query_lon, obs_lat, obs_lon, obs_targets):
    q = query_lat.shape[0]
    k = obs_lat.shape[0]
    kp = ((k + 127) // 128) * 128

    olat2 = jnp.pad(obs_lat, (0, kp - k)).reshape(kp // 128, 128)
    olon2 = jnp.pad(obs_lon, (0, kp - k)).reshape(kp // 128, 128)
    ox2, oy2, oz2 = _features(olat2, olon2)
    ox = ox2.reshape(kp)[:k]
    oy = oy2.reshape(kp)[:k]
    oz = oz2.reshape(kp)[:k]

    qx2, qy2, qz2 = _features(query_lat.reshape(q // 128, 128),
                              query_lon.reshape(q // 128, 128))
    qx = qx2.reshape(q)
    qy = qy2.reshape(q)
    qz = qz2.reshape(q)

    oi, clat, clon, ctgt = _topk_sc(
        jnp.repeat(qx, _LANES), jnp.repeat(qy, _LANES),
        jnp.repeat(qz, _LANES), ox, oy, oz,
        obs_lat, obs_lon, obs_targets)

    dists, indices, targets = _refine(
        query_lat.reshape(q, 1), query_lon.reshape(q, 1),
        clat.reshape(q, _M), clon.reshape(q, _M), ctgt.reshape(q, _M),
        oi.reshape(q, _M))
    return dists, indices, targets


# speedup vs baseline: 4.2969x; 1.1484x over previous
import functools

import jax
import jax.numpy as jnp
from jax import lax
from jax.experimental import pallas as pl
from jax.experimental.pallas import tpu as pltpu
from jax.experimental.pallas import tpu_sc as plsc

_RADIUS = 6371.0
_KNN = 10
_M = 16
_NWORKER = 32
_CHUNK = 20000
_LANES = 16
_G = 10

_SIN_C = [1.0, -1.0 / 6, 1.0 / 120, -1.0 / 5040, 1.0 / 362880,
          -1.0 / 39916800]
_COS_C = [1.0, -0.5, 1.0 / 24, -1.0 / 720, 1.0 / 40320, -1.0 / 3628800,
          1.0 / 479001600]
_ASIN_C = []
_c = 1.0
for _n in range(14):
    if _n:
        _c *= ((2 * _n - 1) / (2 * _n)) * ((2 * _n - 1) / (2 * _n + 1))
    _ASIN_C.append(_c)
del _c, _n


def _poly_u(u, coeffs):
    p = jnp.float32(coeffs[-1])
    for c in reversed(coeffs[:-1]):
        p = p * u + jnp.float32(c)
    return p


def _sinp(t):
    return t * _poly_u(t * t, _SIN_C)


def _cosp(t):
    return _poly_u(t * t, _COS_C)


def _asinp(x):
    return x * _poly_u(x * x, _ASIN_C)


def _feat_body(lat_ref, lon_ref, x_ref, y_ref, z_ref):
    lat = lat_ref[...]
    lon = lon_ref[...]
    cl = _cosp(lat)
    x_ref[...] = _sinp(lat)
    y_ref[...] = cl * _cosp(lon)
    z_ref[...] = cl * _sinp(lon)


def _features(lat2d, lon2d):
    out = jax.ShapeDtypeStruct(lat2d.shape, jnp.float32)
    return pl.pallas_call(_feat_body, out_shape=[out, out, out])(lat2d, lon2d)


def _sc_body(nchunk, qpw,
             qx_h, qy_h, qz_h, ox_h, oy_h, oz_h, olat_h, olon_h, otgt_h,
             oi_h, clat_h, clon_h, ctgt_h,
             qxv, qyv, qzv, oxv, oyv, ozv, bdv, biv, g0, g1, g2, sem):
    wid = lax.axis_index("s") * 2 + lax.axis_index("c")
    qb = wid * qpw
    ncand = qpw * _M

    pltpu.sync_copy(qx_h.at[pl.ds(qb * _LANES, qpw * _LANES)], qxv)
    pltpu.sync_copy(qy_h.at[pl.ds(qb * _LANES, qpw * _LANES)], qyv)
    pltpu.sync_copy(qz_h.at[pl.ds(qb * _LANES, qpw * _LANES)], qzv)

    def init_q(q, carry):
        bdv[pl.ds(q * _M, _M)] = jnp.full((_LANES,), jnp.inf, jnp.float32)
        biv[q >> 3, pl.ds((q & 7) * _M, _M)] = jnp.zeros((_LANES,), jnp.int32)
        return carry

    lax.fori_loop(0, qpw, init_q, 0)

    def chunk_loop(ci, carry):
        pltpu.sync_copy(ox_h.at[pl.ds(ci * _CHUNK, _CHUNK)], oxv)
        pltpu.sync_copy(oy_h.at[pl.ds(ci * _CHUNK, _CHUNK)], oyv)
        pltpu.sync_copy(oz_h.at[pl.ds(ci * _CHUNK, _CHUNK)], ozv)
        ibase = ci * _CHUNK

        def q_loop(qq, carry):
            qa = qq * 2
            qb_ = qa + 1
            qxa = qxv[pl.ds(qa * _LANES, _LANES)]
            qya = qyv[pl.ds(qa * _LANES, _LANES)]
            qza = qzv[pl.ds(qa * _LANES, _LANES)]
            qxb = qxv[pl.ds(qb_ * _LANES, _LANES)]
            qyb = qyv[pl.ds(qb_ * _LANES, _LANES)]
            qzb = qzv[pl.ds(qb_ * _LANES, _LANES)]
            bda0 = bdv[pl.ds(qa * _M, _M)]
            bia0 = biv[qa >> 3, pl.ds((qa & 7) * _M, _M)]
            bdb0 = bdv[pl.ds(qb_ * _M, _M)]
            bib0 = biv[qb_ >> 3, pl.ds((qb_ & 7) * _M, _M)]
            thra0 = jnp.full((_LANES,), bda0[_M - 1], jnp.float32)
            thrb0 = jnp.full((_LANES,), bdb0[_M - 1], jnp.float32)

            def g_body(g, car):
                bda, bia, thra, bdb, bib, thrb = car
                off0 = g * (_LANES * _G)
                d2as = []
                d2bs = []
                for t in range(_G):
                    o = off0 + t * _LANES
                    vx = oxv[pl.ds(o, _LANES)]
                    vy = oyv[pl.ds(o, _LANES)]
                    vz = ozv[pl.ds(o, _LANES)]
                    dxa = vx - qxa
                    dya = vy - qya
                    dza = vz - qza
                    d2as.append(dxa * dxa + dya * dya + dza * dza)
                    dxb = vx - qxb
                    dyb = vy - qyb
                    dzb = vz - qzb
                    d2bs.append(dxb * dxb + dyb * dyb + dzb * dzb)
                mna = d2as[0]
                mnb = d2bs[0]
                for t in range(1, _G):
                    mna = jnp.minimum(mna, d2as[t])
                    mnb = jnp.minimum(mnb, d2bs[t])

                def merge(bd_c, bi_c, d2, m, t):
                    idxv = (ibase + off0 + jnp.int32(t * _LANES)
                            + lax.iota(jnp.int32, _LANES))
                    dm = jnp.where(m, d2, jnp.inf)
                    sd, si = plsc.sort_key_val(dm, idxv, descending=True)
                    lo = jnp.minimum(bd_c, sd)
                    ni = jnp.where(bd_c <= sd, bi_c, si)
                    nd, nid = plsc.sort_key_val(lo, ni)
                    return nd, nid, jnp.full((_LANES,), nd[_M - 1],
                                             jnp.float32)

                def ins_group(bd0_, bi0_, thr0_, d2s):
                    def run():
                        car2 = (bd0_, bi0_, thr0_)
                        for t in range(_G):
                            bd_c, bi_c, thr_c = car2
                            d2 = d2s[t]
                            m = d2 < thr_c
                            nh = plsc.all_reduce_population_count(m)
                            car2 = lax.cond(
                                nh[0] > 0,
                                lambda a=bd_c, b=bi_c, d=d2, mm=m, tt=t:
                                    merge(a, b, d, mm, tt),
                                lambda a=bd_c, b=bi_c, c=thr_c: (a, b, c))
                        return car2
                    return run

                hita = plsc.all_reduce_population_count(mna < thra)
                bda, bia, thra = lax.cond(
                    hita[0] > 0, ins_group(bda, bia, thra, d2as),
                    lambda a=bda, b=bia, c=thra: (a, b, c))
                hitb = plsc.all_reduce_population_count(mnb < thrb)
                bdb, bib, thrb = lax.cond(
                    hitb[0] > 0, ins_group(bdb, bib, thrb, d2bs),
                    lambda a=bdb, b=bib, c=thrb: (a, b, c))
                return bda, bia, thra, bdb, bib, thrb

            out = lax.fori_loop(0, _CHUNK // (_LANES * _G), g_body,
                                (bda0, bia0, thra0, bdb0, bib0, thrb0))
            bda1, bia1, _, bdb1, bib1, _ = out
            bdv[pl.ds(qa * _M, _M)] = bda1
            biv[qa >> 3, pl.ds((qa & 7) * _M, _M)] = bia1
            bdv[pl.ds(qb_ * _M, _M)] = bdb1
            biv[qb_ >> 3, pl.ds((qb_ & 7) * _M, _M)] = bib1
            return carry

        lax.fori_loop(0, qpw // 2, q_loop, 0)
        return carry

    lax.fori_loop(0, nchunk, chunk_loop, 0)

    for p in range(ncand // 128):
        pltpu.async_copy(olat_h.at[biv.at[p]],
                         g0.at[pl.ds(p * 128, 128)], sem).wait()
        pltpu.async_copy(olon_h.at[biv.at[p]],
                         g1.at[pl.ds(p * 128, 128)], sem).wait()
        pltpu.async_copy(otgt_h.at[biv.at[p]],
                         g2.at[pl.ds(p * 128, 128)], sem).wait()

    for p in range(ncand // 128):
        pltpu.sync_copy(biv.at[p], oi_h.at[pl.ds(qb * _M + p * 128, 128)])
    pltpu.sync_copy(g0, clat_h.at[pl.ds(qb * _M, ncand)])
    pltpu.sync_copy(g1, clon_h.at[pl.ds(qb * _M, ncand)])
    pltpu.sync_copy(g2, ctgt_h.at[pl.ds(qb * _M, ncand)])


def _topk_sc(qx, qy, qz, ox, oy, oz, obs_lat, obs_lon, obs_tgt):
    q = qx.shape[0] // _LANES
    k = ox.shape[0]
    assert k % _CHUNK == 0 and q % _NWORKER == 0
    qpw = q // _NWORKER
    ncand = qpw * _M
    mesh = plsc.VectorSubcoreMesh(core_axis_name="c", subcore_axis_name="s")
    f32, i32 = jnp.float32, jnp.int32
    fn = functools.partial(_sc_body, k // _CHUNK, qpw)
    kfn = pl.kernel(
        fn,
        mesh=mesh,
        compiler_params=pltpu.CompilerParams(needs_layout_passes=False),
        out_type=[
            jax.ShapeDtypeStruct((q * _M,), i32),
            jax.ShapeDtypeStruct((q * _M,), f32),
            jax.ShapeDtypeStruct((q * _M,), f32),
            jax.ShapeDtypeStruct((q * _M,), f32),
        ],
        scratch_types=[
            pltpu.VMEM((qpw * _LANES,), f32),
            pltpu.VMEM((qpw * _LANES,), f32),
            pltpu.VMEM((qpw * _LANES,), f32),
            pltpu.VMEM((_CHUNK,), f32),
            pltpu.VMEM((_CHUNK,), f32),
            pltpu.VMEM((_CHUNK,), f32),
            pltpu.VMEM((ncand,), f32),
            pltpu.VMEM((ncand // 128, 128), i32),
            pltpu.VMEM((ncand,), f32),
            pltpu.VMEM((ncand,), f32),
            pltpu.VMEM((ncand,), f32),
            pltpu.SemaphoreType.DMA,
        ],
    )
    return kfn(qx, qy, qz, ox, oy, oz, obs_lat, obs_lon, obs_tgt)


def _refine_body(qlat_ref, qlon_ref, klat_ref, klon_ref, ktgt_ref, kidx_ref,
                 d_ref, i_ref, t_ref):
    ql = qlat_ref[...]
    qn = qlon_ref[...]
    kl = klat_ref[...]
    kn = klon_ref[...]
    kt = ktgt_ref[...]
    ki = kidx_ref[...]

    sdlat = _sinp((ql - kl) * 0.5)
    sdlon = _sinp((qn - kn) * 0.5)
    a = sdlat * sdlat + _cosp(ql) * _cosp(kl) * sdlon * sdlon
    a = jnp.clip(a, 1e-12, 1.0)
    d_all = (2.0 * _RADIUS) * _asinp(jnp.sqrt(a))

    big = jnp.int32(2 ** 30)
    col = lax.broadcasted_iota(jnp.int32, a.shape, 1)
    vals = a
    out_d = jnp.zeros(a.shape, jnp.float32)
    out_i = jnp.zeros(a.shape, jnp.int32)
    out_t = jnp.zeros(a.shape, jnp.float32)
    for j in range(_KNN):
        mn = jnp.min(vals, axis=1, keepdims=True)
        ismn = vals == mn
        pstar = jnp.min(jnp.where(ismn, ki, big), axis=1, keepdims=True)
        pick = ismn & (ki == pstar)
        dj = jnp.sum(jnp.where(pick, d_all, 0.0), axis=1, keepdims=True)
        tj = jnp.sum(jnp.where(pick, kt, 0.0), axis=1, keepdims=True)
        out_d = jnp.where(col == j, dj, out_d)
        out_i = jnp.where(col == j, pstar, out_i)
        out_t = jnp.where(col == j, tj, out_t)
        vals = jnp.where(pick, jnp.inf, vals)

    d_ref[...] = out_d[:, :_KNN]
    i_ref[...] = out_i[:, :_KNN]
    t_ref[...] = out_t[:, :_KNN]


def _refine(qlat, qlon, clat, clon, ctgt, cidx):
    q = qlat.shape[0]
    return pl.pallas_call(
        _refine_body,
        out_shape=[
            jax.ShapeDtypeStruct((q, _KNN), jnp.float32),
            jax.ShapeDtypeStruct((q, _KNN), jnp.int32),
            jax.ShapeDtypeStruct((q, _KNN), jnp.float32),
        ],
    )(qlat, qlon, clat, clon, ctgt, cidx)


def kernel(query_lat, query_lon, obs_lat, obs_lon, obs_targets):
    q = query_lat.shape[0]
    k = obs_lat.shape[0]
    kp = ((k + 127) // 128) * 128

    olat2 = jnp.pad(obs_lat, (0, kp - k)).reshape(kp // 128, 128)
    olon2 = jnp.pad(obs_lon, (0, kp - k)).reshape(kp // 128, 128)
    ox2, oy2, oz2 = _features(olat2, olon2)
    ox = ox2.reshape(kp)[:k]
    oy = oy2.reshape(kp)[:k]
    oz = oz2.reshape(kp)[:k]

    qx2, qy2, qz2 = _features(query_lat.reshape(q // 128, 128),
                              query_lon.reshape(q // 128, 128))
    qx = qx2.reshape(q)
    qy = qy2.reshape(q)
    qz = qz2.reshape(q)

    oi, clat, clon, ctgt = _topk_sc(
        jnp.repeat(qx, _LANES), jnp.repeat(qy, _LANES),
        jnp.repeat(qz, _LANES), ox, oy, oz,
        obs_lat, obs_lon, obs_targets)

    dists, indices, targets = _refine(
        query_lat.reshape(q, 1), query_lon.reshape(q, 1),
        clat.reshape(q, _M), clon.reshape(q, _M), ctgt.reshape(q, _M),
        oi.reshape(q, _M))
    return dists, indices, targets


# speedup vs baseline: 4.4072x; 1.0257x over previous
import functools

import jax
import jax.numpy as jnp
from jax import lax
from jax.experimental import pallas as pl
from jax.experimental.pallas import tpu as pltpu
from jax.experimental.pallas import tpu_sc as plsc

_RADIUS = 6371.0
_KNN = 10
_M = 16
_NWORKER = 32
_CHUNK = 20000
_LANES = 16
_G = 10
_P = 4

_SIN_C = [1.0, -1.0 / 6, 1.0 / 120, -1.0 / 5040, 1.0 / 362880,
          -1.0 / 39916800]
_COS_C = [1.0, -0.5, 1.0 / 24, -1.0 / 720, 1.0 / 40320, -1.0 / 3628800,
          1.0 / 479001600]
_ASIN_C = []
_c = 1.0
for _n in range(14):
    if _n:
        _c *= ((2 * _n - 1) / (2 * _n)) * ((2 * _n - 1) / (2 * _n + 1))
    _ASIN_C.append(_c)
del _c, _n


def _poly_u(u, coeffs):
    p = jnp.float32(coeffs[-1])
    for c in reversed(coeffs[:-1]):
        p = p * u + jnp.float32(c)
    return p


def _sinp(t):
    return t * _poly_u(t * t, _SIN_C)


def _cosp(t):
    return _poly_u(t * t, _COS_C)


def _asinp(x):
    return x * _poly_u(x * x, _ASIN_C)


def _feat_body(lat_ref, lon_ref, x_ref, y_ref, z_ref):
    lat = lat_ref[...]
    lon = lon_ref[...]
    cl = _cosp(lat)
    x_ref[...] = _sinp(lat)
    y_ref[...] = cl * _cosp(lon)
    z_ref[...] = cl * _sinp(lon)


def _features(lat2d, lon2d):
    out = jax.ShapeDtypeStruct(lat2d.shape, jnp.float32)
    return pl.pallas_call(_feat_body, out_shape=[out, out, out])(lat2d, lon2d)


def _sc_body(nchunk, qpw,
             qx_h, qy_h, qz_h, ox_h, oy_h, oz_h, olat_h, olon_h, otgt_h,
             oi_h, clat_h, clon_h, ctgt_h,
             qxv, qyv, qzv, oxv, oyv, ozv, bdv, biv, g0, g1, g2, sem):
    wid = lax.axis_index("s") * 2 + lax.axis_index("c")
    qb = wid * qpw
    ncand = qpw * _M

    pltpu.sync_copy(qx_h.at[pl.ds(qb * _LANES, qpw * _LANES)], qxv)
    pltpu.sync_copy(qy_h.at[pl.ds(qb * _LANES, qpw * _LANES)], qyv)
    pltpu.sync_copy(qz_h.at[pl.ds(qb * _LANES, qpw * _LANES)], qzv)

    def init_q(q, carry):
        bdv[pl.ds(q * _M, _M)] = jnp.full((_LANES,), jnp.inf, jnp.float32)
        biv[q >> 3, pl.ds((q & 7) * _M, _M)] = jnp.zeros((_LANES,), jnp.int32)
        return carry

    lax.fori_loop(0, qpw, init_q, 0)

    def chunk_loop(ci, carry):
        pltpu.sync_copy(ox_h.at[pl.ds(ci * _CHUNK, _CHUNK)], oxv)
        pltpu.sync_copy(oy_h.at[pl.ds(ci * _CHUNK, _CHUNK)], oyv)
        pltpu.sync_copy(oz_h.at[pl.ds(ci * _CHUNK, _CHUNK)], ozv)
        ibase = ci * _CHUNK

        def q_loop(qq, carry):
            qf = []
            st = []
            for j in range(_P):
                q = qq * _P + j
                qf.append((qxv[pl.ds(q * _LANES, _LANES)],
                           qyv[pl.ds(q * _LANES, _LANES)],
                           qzv[pl.ds(q * _LANES, _LANES)]))
                bd0 = bdv[pl.ds(q * _M, _M)]
                bi0 = biv[q >> 3, pl.ds((q & 7) * _M, _M)]
                st += [bd0, bi0, jnp.full((_LANES,), bd0[_M - 1],
                                          jnp.float32)]

            def g_body(g, car):
                off0 = g * (_LANES * _G)
                d2s = [[] for _ in range(_P)]
                for t in range(_G):
                    o = off0 + t * _LANES
                    vx = oxv[pl.ds(o, _LANES)]
                    vy = oyv[pl.ds(o, _LANES)]
                    vz = ozv[pl.ds(o, _LANES)]
                    for j in range(_P):
                        dx = vx - qf[j][0]
                        dy = vy - qf[j][1]
                        dz = vz - qf[j][2]
                        d2s[j].append(dx * dx + dy * dy + dz * dz)
                mns = []
                for j in range(_P):
                    mn = d2s[j][0]
                    for t in range(1, _G):
                        mn = jnp.minimum(mn, d2s[j][t])
                    mns.append(mn)

                def merge(bd_c, bi_c, d2, m, t):
                    idxv = (ibase + off0 + jnp.int32(t * _LANES)
                            + lax.iota(jnp.int32, _LANES))
                    dm = jnp.where(m, d2, jnp.inf)
                    sd, si = plsc.sort_key_val(dm, idxv, descending=True)
                    lo = jnp.minimum(bd_c, sd)
                    ni = jnp.where(bd_c <= sd, bi_c, si)
                    nd, nid = plsc.sort_key_val(lo, ni)
                    return nd, nid, jnp.full((_LANES,), nd[_M - 1],
                                             jnp.float32)

                def ins_group(bd0_, bi0_, thr0_, d2g):
                    def run():
                        car2 = (bd0_, bi0_, thr0_)
                        for t in range(_G):
                            bd_c, bi_c, thr_c = car2
                            d2 = d2g[t]
                            m = d2 < thr_c
                            nh = plsc.all_reduce_population_count(m)
                            car2 = lax.cond(
                                nh[0] > 0,
                                lambda a=bd_c, b=bi_c, d=d2, mm=m, tt=t:
                                    merge(a, b, d, mm, tt),
                                lambda a=bd_c, b=bi_c, c=thr_c: (a, b, c))
                        return car2
                    return run

                out = []
                for j in range(_P):
                    bd_j, bi_j, thr_j = car[3 * j], car[3 * j + 1], car[3 * j + 2]
                    hit = plsc.all_reduce_population_count(mns[j] < thr_j)
                    bd_j, bi_j, thr_j = lax.cond(
                        hit[0] > 0, ins_group(bd_j, bi_j, thr_j, d2s[j]),
                        lambda a=bd_j, b=bi_j, c=thr_j: (a, b, c))
                    out += [bd_j, bi_j, thr_j]
                return tuple(out)

            fin = lax.fori_loop(0, _CHUNK // (_LANES * _G), g_body,
                                tuple(st))
            for j in range(_P):
                q = qq * _P + j
                bdv[pl.ds(q * _M, _M)] = fin[3 * j]
                biv[q >> 3, pl.ds((q & 7) * _M, _M)] = fin[3 * j + 1]
            return carry

        lax.fori_loop(0, qpw // _P, q_loop, 0)
        return carry

    lax.fori_loop(0, nchunk, chunk_loop, 0)

    for p in range(ncand // 128):
        pltpu.async_copy(olat_h.at[biv.at[p]],
                         g0.at[pl.ds(p * 128, 128)], sem).wait()
        pltpu.async_copy(olon_h.at[biv.at[p]],
                         g1.at[pl.ds(p * 128, 128)], sem).wait()
        pltpu.async_copy(otgt_h.at[biv.at[p]],
                         g2.at[pl.ds(p * 128, 128)], sem).wait()

    for p in range(ncand // 128):
        pltpu.sync_copy(biv.at[p], oi_h.at[pl.ds(qb * _M + p * 128, 128)])
    pltpu.sync_copy(g0, clat_h.at[pl.ds(qb * _M, ncand)])
    pltpu.sync_copy(g1, clon_h.at[pl.ds(qb * _M, ncand)])
    pltpu.sync_copy(g2, ctgt_h.at[pl.ds(qb * _M, ncand)])


def _topk_sc(qx, qy, qz, ox, oy, oz, obs_lat, obs_lon, obs_tgt):
    q = qx.shape[0] // _LANES
    k = ox.shape[0]
    assert k % _CHUNK == 0 and q % _NWORKER == 0
    qpw = q // _NWORKER
    ncand = qpw * _M
    mesh = plsc.VectorSubcoreMesh(core_axis_name="c", subcore_axis_name="s")
    f32, i32 = jnp.float32, jnp.int32
    fn = functools.partial(_sc_body, k // _CHUNK, qpw)
    kfn = pl.kernel(
        fn,
        mesh=mesh,
        compiler_params=pltpu.CompilerParams(needs_layout_passes=False),
        out_type=[
            jax.ShapeDtypeStruct((q * _M,), i32),
            jax.ShapeDtypeStruct((q * _M,), f32),
            jax.ShapeDtypeStruct((q * _M,), f32),
            jax.ShapeDtypeStruct((q * _M,), f32),
        ],
        scratch_types=[
            pltpu.VMEM((qpw * _LANES,), f32),
            pltpu.VMEM((qpw * _LANES,), f32),
            pltpu.VMEM((qpw * _LANES,), f32),
            pltpu.VMEM((_CHUNK,), f32),
            pltpu.VMEM((_CHUNK,), f32),
            pltpu.VMEM((_CHUNK,), f32),
            pltpu.VMEM((ncand,), f32),
            pltpu.VMEM((ncand // 128, 128), i32),
            pltpu.VMEM((ncand,), f32),
            pltpu.VMEM((ncand,), f32),
            pltpu.VMEM((ncand,), f32),
            pltpu.SemaphoreType.DMA,
        ],
    )
    return kfn(qx, qy, qz, ox, oy, oz, obs_lat, obs_lon, obs_tgt)


def _refine_body(qlat_ref, qlon_ref, klat_ref, klon_ref, ktgt_ref, kidx_ref,
                 d_ref, i_ref, t_ref):
    ql = qlat_ref[...]
    qn = qlon_ref[...]
    kl = klat_ref[...]
    kn = klon_ref[...]
    kt = ktgt_ref[...]
    ki = kidx_ref[...]

    sdlat = _sinp((ql - kl) * 0.5)
    sdlon = _sinp((qn - kn) * 0.5)
    a = sdlat * sdlat + _cosp(ql) * _cosp(kl) * sdlon * sdlon
    a = jnp.clip(a, 1e-12, 1.0)
    d_all = (2.0 * _RADIUS) * _asinp(jnp.sqrt(a))

    big = jnp.int32(2 ** 30)
    col = lax.broadcasted_iota(jnp.int32, a.shape, 1)
    vals = a
    out_d = jnp.zeros(a.shape, jnp.float32)
    out_i = jnp.zeros(a.shape, jnp.int32)
    out_t = jnp.zeros(a.shape, jnp.float32)
    for j in range(_KNN):
        mn = jnp.min(vals, axis=1, keepdims=True)
        ismn = vals == mn
        pstar = jnp.min(jnp.where(ismn, ki, big), axis=1, keepdims=True)
        pick = ismn & (ki == pstar)
        dj = jnp.sum(jnp.where(pick, d_all, 0.0), axis=1, keepdims=True)
        tj = jnp.sum(jnp.where(pick, kt, 0.0), axis=1, keepdims=True)
        out_d = jnp.where(col == j, dj, out_d)
        out_i = jnp.where(col == j, pstar, out_i)
        out_t = jnp.where(col == j, tj, out_t)
        vals = jnp.where(pick, jnp.inf, vals)

    d_ref[...] = out_d[:, :_KNN]
    i_ref[...] = out_i[:, :_KNN]
    t_ref[...] = out_t[:, :_KNN]


def _refine(qlat, qlon, clat, clon, ctgt, cidx):
    q = qlat.shape[0]
    return pl.pallas_call(
        _refine_body,
        out_shape=[
            jax.ShapeDtypeStruct((q, _KNN), jnp.float32),
            jax.ShapeDtypeStruct((q, _KNN), jnp.int32),
            jax.ShapeDtypeStruct((q, _KNN), jnp.float32),
        ],
    )(qlat, qlon, clat, clon, ctgt, cidx)


def kernel(query_lat, query_lon, obs_lat, obs_lon, obs_targets):
    q = query_lat.shape[0]
    k = obs_lat.shape[0]
    kp = ((k + 127) // 128) * 128

    olat2 = jnp.pad(obs_lat, (0, kp - k)).reshape(kp // 128, 128)
    olon2 = jnp.pad(obs_lon, (0, kp - k)).reshape(kp // 128, 128)
    ox2, oy2, oz2 = _features(olat2, olon2)
    ox = ox2.reshape(kp)[:k]
    oy = oy2.reshape(kp)[:k]
    oz = oz2.reshape(kp)[:k]

    qx2, qy2, qz2 = _features(query_lat.reshape(q // 128, 128),
                              query_lon.reshape(q // 128, 128))
    qx = qx2.reshape(q)
    qy = qy2.reshape(q)
    qz = qz2.reshape(q)

    oi, clat, clon, ctgt = _topk_sc(
        jnp.repeat(qx, _LANES), jnp.repeat(qy, _LANES),
        jnp.repeat(qz, _LANES), ox, oy, oz,
        obs_lat, obs_lon, obs_targets)

    dists, indices, targets = _refine(
        query_lat.reshape(q, 1), query_lon.reshape(q, 1),
        clat.reshape(q, _M), clon.reshape(q, _M), ctgt.reshape(q, _M),
        oi.reshape(q, _M))
    return dists, indices, targets


# speedup vs baseline: 4.9606x; 1.1256x over previous
import functools

import jax
import jax.numpy as jnp
from jax import lax
from jax.experimental import pallas as pl
from jax.experimental.pallas import tpu as pltpu
from jax.experimental.pallas import tpu_sc as plsc

_RADIUS = 6371.0
_KNN = 10
_M = 16
_NWORKER = 32
_CHUNK = 20000
_LANES = 16
_G = 10
_P = 4

_SIN_C = [1.0, -1.0 / 6, 1.0 / 120, -1.0 / 5040, 1.0 / 362880,
          -1.0 / 39916800]
_COS_C = [1.0, -0.5, 1.0 / 24, -1.0 / 720, 1.0 / 40320, -1.0 / 3628800,
          1.0 / 479001600]
_ASIN_C = []
_c = 1.0
for _n in range(14):
    if _n:
        _c *= ((2 * _n - 1) / (2 * _n)) * ((2 * _n - 1) / (2 * _n + 1))
    _ASIN_C.append(_c)
del _c, _n


def _poly_u(u, coeffs):
    p = jnp.float32(coeffs[-1])
    for c in reversed(coeffs[:-1]):
        p = p * u + jnp.float32(c)
    return p


def _sinp(t):
    return t * _poly_u(t * t, _SIN_C)


def _cosp(t):
    return _poly_u(t * t, _COS_C)


def _asinp(x):
    return x * _poly_u(x * x, _ASIN_C)


def _feat_body(lat_ref, lon_ref, x_ref, y_ref, z_ref):
    lat = lat_ref[...]
    lon = lon_ref[...]
    cl = _cosp(lat)
    x_ref[...] = _sinp(lat)
    y_ref[...] = cl * _cosp(lon)
    z_ref[...] = cl * _sinp(lon)


def _features(lat2d, lon2d):
    out = jax.ShapeDtypeStruct(lat2d.shape, jnp.float32)
    return pl.pallas_call(_feat_body, out_shape=[out, out, out])(lat2d, lon2d)


def _sc_body(nchunk, qpw,
             qx_h, qy_h, qz_h, ox_h, oy_h, oz_h, olat_h, olon_h, otgt_h,
             oi_h, clat_h, clon_h, ctgt_h,
             qxv, qyv, qzv, oxv, oyv, ozv, bdv, biv, g0, g1, g2, sem):
    wid = lax.axis_index("s") * 2 + lax.axis_index("c")
    qb = wid * qpw
    ncand = qpw * _M

    pltpu.sync_copy(qx_h.at[pl.ds(qb * _LANES, qpw * _LANES)], qxv)
    pltpu.sync_copy(qy_h.at[pl.ds(qb * _LANES, qpw * _LANES)], qyv)
    pltpu.sync_copy(qz_h.at[pl.ds(qb * _LANES, qpw * _LANES)], qzv)

    def init_q(q, carry):
        bdv[pl.ds(q * _M, _M)] = jnp.full((_LANES,), jnp.inf, jnp.float32)
        biv[q >> 3, pl.ds((q & 7) * _M, _M)] = jnp.zeros((_LANES,), jnp.int32)
        return carry

    lax.fori_loop(0, qpw, init_q, 0)

    def chunk_loop(ci, carry):
        pltpu.sync_copy(ox_h.at[pl.ds(ci * _CHUNK, _CHUNK)], oxv)
        pltpu.sync_copy(oy_h.at[pl.ds(ci * _CHUNK, _CHUNK)], oyv)
        pltpu.sync_copy(oz_h.at[pl.ds(ci * _CHUNK, _CHUNK)], ozv)
        ibase = ci * _CHUNK

        def q_loop(qq, carry):
            qf = []
            st = []
            for j in range(_P):
                q = qq * _P + j
                qf.append((qxv[pl.ds(q * _LANES, _LANES)],
                           qyv[pl.ds(q * _LANES, _LANES)],
                           qzv[pl.ds(q * _LANES, _LANES)]))
                bd0 = bdv[pl.ds(q * _M, _M)]
                bi0 = biv[q >> 3, pl.ds((q & 7) * _M, _M)]
                st += [bd0, bi0, jnp.full((_LANES,), bd0[_M - 1],
                                          jnp.float32)]

            def g_body(g, car):
                off0 = g * (_LANES * _G)
                d2s = [[] for _ in range(_P)]
                for t in range(_G):
                    o = off0 + t * _LANES
                    vx = oxv[pl.ds(o, _LANES)]
                    vy = oyv[pl.ds(o, _LANES)]
                    vz = ozv[pl.ds(o, _LANES)]
                    for j in range(_P):
                        dx = vx - qf[j][0]
                        dy = vy - qf[j][1]
                        dz = vz - qf[j][2]
                        d2s[j].append(dx * dx + dy * dy + dz * dz)
                mns = []
                for j in range(_P):
                    mn = d2s[j][0]
                    for t in range(1, _G):
                        mn = jnp.minimum(mn, d2s[j][t])
                    mns.append(mn)

                def merge(bd_c, bi_c, d2, m, t):
                    idxv = (ibase + off0 + jnp.int32(t * _LANES)
                            + lax.iota(jnp.int32, _LANES))
                    dm = jnp.where(m, d2, jnp.inf)
                    sd, si = plsc.sort_key_val(dm, idxv, descending=True)
                    lo = jnp.minimum(bd_c, sd)
                    ni = jnp.where(bd_c <= sd, bi_c, si)
                    nd, nid = plsc.sort_key_val(lo, ni)
                    return nd, nid, jnp.full((_LANES,), nd[_M - 1],
                                             jnp.float32)

                def ins_group(bd0_, bi0_, thr0_, d2g):
                    def run():
                        car2 = (bd0_, bi0_, thr0_)
                        for t in range(_G):
                            bd_c, bi_c, thr_c = car2
                            d2 = d2g[t]
                            m = d2 < thr_c
                            nh = plsc.all_reduce_population_count(m)
                            car2 = lax.cond(
                                nh[0] > 0,
                                lambda a=bd_c, b=bi_c, d=d2, mm=m, tt=t:
                                    merge(a, b, d, mm, tt),
                                lambda a=bd_c, b=bi_c, c=thr_c: (a, b, c))
                        return car2
                    return run

                hit_any = mns[0] < car[2]
                for j in range(1, _P):
                    hit_any = hit_any | (mns[j] < car[3 * j + 2])
                nh_any = plsc.all_reduce_population_count(hit_any)

                def fire():
                    out = []
                    for j in range(_P):
                        bd_j = car[3 * j]
                        bi_j = car[3 * j + 1]
                        thr_j = car[3 * j + 2]
                        h = plsc.all_reduce_population_count(
                            mns[j] < thr_j)
                        bd_j, bi_j, thr_j = lax.cond(
                            h[0] > 0, ins_group(bd_j, bi_j, thr_j, d2s[j]),
                            lambda a=bd_j, b=bi_j, c=thr_j: (a, b, c))
                        out += [bd_j, bi_j, thr_j]
                    return tuple(out)

                return lax.cond(nh_any[0] > 0, fire, lambda: tuple(car))

            fin = lax.fori_loop(0, _CHUNK // (_LANES * _G), g_body,
                                tuple(st))
            for j in range(_P):
                q = qq * _P + j
                bdv[pl.ds(q * _M, _M)] = fin[3 * j]
                biv[q >> 3, pl.ds((q & 7) * _M, _M)] = fin[3 * j + 1]
            return carry

        lax.fori_loop(0, qpw // _P, q_loop, 0)
        return carry

    lax.fori_loop(0, nchunk, chunk_loop, 0)

    for p in range(ncand // 128):
        pltpu.async_copy(olat_h.at[biv.at[p]],
                         g0.at[pl.ds(p * 128, 128)], sem).wait()
        pltpu.async_copy(olon_h.at[biv.at[p]],
                         g1.at[pl.ds(p * 128, 128)], sem).wait()
        pltpu.async_copy(otgt_h.at[biv.at[p]],
                         g2.at[pl.ds(p * 128, 128)], sem).wait()

    for p in range(ncand // 128):
        pltpu.sync_copy(biv.at[p], oi_h.at[pl.ds(qb * _M + p * 128, 128)])
    pltpu.sync_copy(g0, clat_h.at[pl.ds(qb * _M, ncand)])
    pltpu.sync_copy(g1, clon_h.at[pl.ds(qb * _M, ncand)])
    pltpu.sync_copy(g2, ctgt_h.at[pl.ds(qb * _M, ncand)])


def _topk_sc(qx, qy, qz, ox, oy, oz, obs_lat, obs_lon, obs_tgt):
    q = qx.shape[0] // _LANES
    k = ox.shape[0]
    assert k % _CHUNK == 0 and q % _NWORKER == 0
    qpw = q // _NWORKER
    ncand = qpw * _M
    mesh = plsc.VectorSubcoreMesh(core_axis_name="c", subcore_axis_name="s")
    f32, i32 = jnp.float32, jnp.int32
    fn = functools.partial(_sc_body, k // _CHUNK, qpw)
    kfn = pl.kernel(
        fn,
        mesh=mesh,
        compiler_params=pltpu.CompilerParams(needs_layout_passes=False),
        out_type=[
            jax.ShapeDtypeStruct((q * _M,), i32),
            jax.ShapeDtypeStruct((q * _M,), f32),
            jax.ShapeDtypeStruct((q * _M,), f32),
            jax.ShapeDtypeStruct((q * _M,), f32),
        ],
        scratch_types=[
            pltpu.VMEM((qpw * _LANES,), f32),
            pltpu.VMEM((qpw * _LANES,), f32),
            pltpu.VMEM((qpw * _LANES,), f32),
            pltpu.VMEM((_CHUNK,), f32),
            pltpu.VMEM((_CHUNK,), f32),
            pltpu.VMEM((_CHUNK,), f32),
            pltpu.VMEM((ncand,), f32),
            pltpu.VMEM((ncand // 128, 128), i32),
            pltpu.VMEM((ncand,), f32),
            pltpu.VMEM((ncand,), f32),
            pltpu.VMEM((ncand,), f32),
            pltpu.SemaphoreType.DMA,
        ],
    )
    return kfn(qx, qy, qz, ox, oy, oz, obs_lat, obs_lon, obs_tgt)


def _refine_body(qlat_ref, qlon_ref, klat_ref, klon_ref, ktgt_ref, kidx_ref,
                 d_ref, i_ref, t_ref):
    ql = qlat_ref[...]
    qn = qlon_ref[...]
    kl = klat_ref[...]
    kn = klon_ref[...]
    kt = ktgt_ref[...]
    ki = kidx_ref[...]

    sdlat = _sinp((ql - kl) * 0.5)
    sdlon = _sinp((qn - kn) * 0.5)
    a = sdlat * sdlat + _cosp(ql) * _cosp(kl) * sdlon * sdlon
    a = jnp.clip(a, 1e-12, 1.0)
    d_all = (2.0 * _RADIUS) * _asinp(jnp.sqrt(a))

    big = jnp.int32(2 ** 30)
    col = lax.broadcasted_iota(jnp.int32, a.shape, 1)
    vals = a
    out_d = jnp.zeros(a.shape, jnp.float32)
    out_i = jnp.zeros(a.shape, jnp.int32)
    out_t = jnp.zeros(a.shape, jnp.float32)
    for j in range(_KNN):
        mn = jnp.min(vals, axis=1, keepdims=True)
        ismn = vals == mn
        pstar = jnp.min(jnp.where(ismn, ki, big), axis=1, keepdims=True)
        pick = ismn & (ki == pstar)
        dj = jnp.sum(jnp.where(pick, d_all, 0.0), axis=1, keepdims=True)
        tj = jnp.sum(jnp.where(pick, kt, 0.0), axis=1, keepdims=True)
        out_d = jnp.where(col == j, dj, out_d)
        out_i = jnp.where(col == j, pstar, out_i)
        out_t = jnp.where(col == j, tj, out_t)
        vals = jnp.where(pick, jnp.inf, vals)

    d_ref[...] = out_d[:, :_KNN]
    i_ref[...] = out_i[:, :_KNN]
    t_ref[...] = out_t[:, :_KNN]


def _refine(qlat, qlon, clat, clon, ctgt, cidx):
    q = qlat.shape[0]
    return pl.pallas_call(
        _refine_body,
        out_shape=[
            jax.ShapeDtypeStruct((q, _KNN), jnp.float32),
            jax.ShapeDtypeStruct((q, _KNN), jnp.int32),
            jax.ShapeDtypeStruct((q, _KNN), jnp.float32),
        ],
    )(qlat, qlon, clat, clon, ctgt, cidx)


def kernel(query_lat, query_lon, obs_lat, obs_lon, obs_targets):
    q = query_lat.shape[0]
    k = obs_lat.shape[0]
    kp = ((k + 127) // 128) * 128

    olat2 = jnp.pad(obs_lat, (0, kp - k)).reshape(kp // 128, 128)
    olon2 = jnp.pad(obs_lon, (0, kp - k)).reshape(kp // 128, 128)
    ox2, oy2, oz2 = _features(olat2, olon2)
    ox = ox2.reshape(kp)[:k]
    oy = oy2.reshape(kp)[:k]
    oz = oz2.reshape(kp)[:k]

    qx2, qy2, qz2 = _features(query_lat.reshape(q // 128, 128),
                              query_lon.reshape(q // 128, 128))
    qx = qx2.reshape(q)
    qy = qy2.reshape(q)
    qz = qz2.reshape(q)

    oi, clat, clon, ctgt = _topk_sc(
        jnp.repeat(qx, _LANES), jnp.repeat(qy, _LANES),
        jnp.repeat(qz, _LANES), ox, oy, oz,
        obs_lat, obs_lon, obs_targets)

    dists, indices, targets = _refine(
        query_lat.reshape(q, 1), query_lon.reshape(q, 1),
        clat.reshape(q, _M), clon.reshape(q, _M), ctgt.reshape(q, _M),
        oi.reshape(q, _M))
    return dists, indices, targets


# speedup vs baseline: 5.2773x; 1.0639x over previous
import functools

import jax
import jax.numpy as jnp
from jax import lax
from jax.experimental import pallas as pl
from jax.experimental.pallas import tpu as pltpu
from jax.experimental.pallas import tpu_sc as plsc

_RADIUS = 6371.0
_KNN = 10
_M = 16
_NWORKER = 32
_CHUNK = 20000
_LANES = 16
_G = 10
_P = 4

_SIN_C = [1.0, -1.0 / 6, 1.0 / 120, -1.0 / 5040, 1.0 / 362880,
          -1.0 / 39916800]
_COS_C = [1.0, -0.5, 1.0 / 24, -1.0 / 720, 1.0 / 40320, -1.0 / 3628800,
          1.0 / 479001600]
_ASIN_C = []
_c = 1.0
for _n in range(14):
    if _n:
        _c *= ((2 * _n - 1) / (2 * _n)) * ((2 * _n - 1) / (2 * _n + 1))
    _ASIN_C.append(_c)
del _c, _n


def _poly_u(u, coeffs):
    p = jnp.float32(coeffs[-1])
    for c in reversed(coeffs[:-1]):
        p = p * u + jnp.float32(c)
    return p


def _sinp(t):
    return t * _poly_u(t * t, _SIN_C)


def _cosp(t):
    return _poly_u(t * t, _COS_C)


def _asinp(x):
    return x * _poly_u(x * x, _ASIN_C)


def _feat_body(lat_ref, lon_ref, x_ref, y_ref, z_ref):
    lat = lat_ref[...]
    lon = lon_ref[...]
    cl = _cosp(lat)
    x_ref[...] = _sinp(lat)
    y_ref[...] = cl * _cosp(lon)
    z_ref[...] = cl * _sinp(lon)


def _features(lat2d, lon2d):
    out = jax.ShapeDtypeStruct(lat2d.shape, jnp.float32)
    return pl.pallas_call(_feat_body, out_shape=[out, out, out])(lat2d, lon2d)


def _sc_body(nchunk, qpw,
             qx_h, qy_h, qz_h, ox_h, oy_h, oz_h, olat_h, olon_h, otgt_h,
             oi_h, clat_h, clon_h, ctgt_h,
             qxv, qyv, qzv, oxv, oyv, ozv, bdv, biv, g0, g1, g2, sem):
    wid = lax.axis_index("s") * 2 + lax.axis_index("c")
    qb = wid * qpw
    ncand = qpw * _M

    pltpu.sync_copy(qx_h.at[pl.ds(qb * _LANES, qpw * _LANES)], qxv)
    pltpu.sync_copy(qy_h.at[pl.ds(qb * _LANES, qpw * _LANES)], qyv)
    pltpu.sync_copy(qz_h.at[pl.ds(qb * _LANES, qpw * _LANES)], qzv)

    def init_q(q, carry):
        bdv[pl.ds(q * _M, _M)] = jnp.full((_LANES,), jnp.inf, jnp.float32)
        biv[q >> 3, pl.ds((q & 7) * _M, _M)] = jnp.zeros((_LANES,), jnp.int32)
        return carry

    lax.fori_loop(0, qpw, init_q, 0)

    def chunk_loop(ci, carry):
        pltpu.sync_copy(ox_h.at[pl.ds(ci * _CHUNK, _CHUNK)], oxv)
        pltpu.sync_copy(oy_h.at[pl.ds(ci * _CHUNK, _CHUNK)], oyv)
        pltpu.sync_copy(oz_h.at[pl.ds(ci * _CHUNK, _CHUNK)], ozv)
        ibase = ci * _CHUNK

        def q_loop(qq, carry):
            qf = []
            st = []
            for j in range(_P):
                q = qq * _P + j
                qf.append((qxv[pl.ds(q * _LANES, _LANES)],
                           qyv[pl.ds(q * _LANES, _LANES)],
                           qzv[pl.ds(q * _LANES, _LANES)]))
                bd0 = bdv[pl.ds(q * _M, _M)]
                bi0 = biv[q >> 3, pl.ds((q & 7) * _M, _M)]
                st += [bd0, bi0, jnp.full((_LANES,), bd0[_M - 1],
                                          jnp.float32)]

            def g_body(g, car):
                off0 = g * (_LANES * _G)
                d2s = [[] for _ in range(_P)]
                for t in range(_G):
                    o = off0 + t * _LANES
                    vx = oxv[pl.ds(o, _LANES)]
                    vy = oyv[pl.ds(o, _LANES)]
                    vz = ozv[pl.ds(o, _LANES)]
                    for j in range(_P):
                        d2s[j].append(-(vx * qf[j][0] + vy * qf[j][1]
                                        + vz * qf[j][2]))
                mns = []
                for j in range(_P):
                    mn = d2s[j][0]
                    for t in range(1, _G):
                        mn = jnp.minimum(mn, d2s[j][t])
                    mns.append(mn)

                def merge(bd_c, bi_c, d2, m, t):
                    idxv = (ibase + off0 + jnp.int32(t * _LANES)
                            + lax.iota(jnp.int32, _LANES))
                    dm = jnp.where(m, d2, jnp.inf)
                    sd, si = plsc.sort_key_val(dm, idxv, descending=True)
                    lo = jnp.minimum(bd_c, sd)
                    ni = jnp.where(bd_c <= sd, bi_c, si)
                    nd, nid = plsc.sort_key_val(lo, ni)
                    return nd, nid, jnp.full((_LANES,), nd[_M - 1],
                                             jnp.float32)

                def ins_group(bd0_, bi0_, thr0_, d2g):
                    def run():
                        car2 = (bd0_, bi0_, thr0_)
                        for t in range(_G):
                            bd_c, bi_c, thr_c = car2
                            d2 = d2g[t]
                            m = d2 < thr_c
                            nh = plsc.all_reduce_population_count(m)
                            car2 = lax.cond(
                                nh[0] > 0,
                                lambda a=bd_c, b=bi_c, d=d2, mm=m, tt=t:
                                    merge(a, b, d, mm, tt),
                                lambda a=bd_c, b=bi_c, c=thr_c: (a, b, c))
                        return car2
                    return run

                hit_any = mns[0] < car[2]
                for j in range(1, _P):
                    hit_any = hit_any | (mns[j] < car[3 * j + 2])
                nh_any = plsc.all_reduce_population_count(hit_any)

                def fire():
                    out = []
                    for j in range(_P):
                        bd_j = car[3 * j]
                        bi_j = car[3 * j + 1]
                        thr_j = car[3 * j + 2]
                        h = plsc.all_reduce_population_count(
                            mns[j] < thr_j)
                        bd_j, bi_j, thr_j = lax.cond(
                            h[0] > 0, ins_group(bd_j, bi_j, thr_j, d2s[j]),
                            lambda a=bd_j, b=bi_j, c=thr_j: (a, b, c))
                        out += [bd_j, bi_j, thr_j]
                    return tuple(out)

                return lax.cond(nh_any[0] > 0, fire, lambda: tuple(car))

            fin = lax.fori_loop(0, _CHUNK // (_LANES * _G), g_body,
                                tuple(st))
            for j in range(_P):
                q = qq * _P + j
                bdv[pl.ds(q * _M, _M)] = fin[3 * j]
                biv[q >> 3, pl.ds((q & 7) * _M, _M)] = fin[3 * j + 1]
            return carry

        lax.fori_loop(0, qpw // _P, q_loop, 0)
        return carry

    lax.fori_loop(0, nchunk, chunk_loop, 0)

    for p in range(ncand // 128):
        pltpu.async_copy(olat_h.at[biv.at[p]],
                         g0.at[pl.ds(p * 128, 128)], sem).wait()
        pltpu.async_copy(olon_h.at[biv.at[p]],
                         g1.at[pl.ds(p * 128, 128)], sem).wait()
        pltpu.async_copy(otgt_h.at[biv.at[p]],
                         g2.at[pl.ds(p * 128, 128)], sem).wait()

    for p in range(ncand // 128):
        pltpu.sync_copy(biv.at[p], oi_h.at[pl.ds(qb * _M + p * 128, 128)])
    pltpu.sync_copy(g0, clat_h.at[pl.ds(qb * _M, ncand)])
    pltpu.sync_copy(g1, clon_h.at[pl.ds(qb * _M, ncand)])
    pltpu.sync_copy(g2, ctgt_h.at[pl.ds(qb * _M, ncand)])


def _topk_sc(qx, qy, qz, ox, oy, oz, obs_lat, obs_lon, obs_tgt):
    q = qx.shape[0] // _LANES
    k = ox.shape[0]
    assert k % _CHUNK == 0 and q % _NWORKER == 0
    qpw = q // _NWORKER
    ncand = qpw * _M
    mesh = plsc.VectorSubcoreMesh(core_axis_name="c", subcore_axis_name="s")
    f32, i32 = jnp.float32, jnp.int32
    fn = functools.partial(_sc_body, k // _CHUNK, qpw)
    kfn = pl.kernel(
        fn,
        mesh=mesh,
        compiler_params=pltpu.CompilerParams(needs_layout_passes=False),
        out_type=[
            jax.ShapeDtypeStruct((q * _M,), i32),
            jax.ShapeDtypeStruct((q * _M,), f32),
            jax.ShapeDtypeStruct((q * _M,), f32),
            jax.ShapeDtypeStruct((q * _M,), f32),
        ],
        scratch_types=[
            pltpu.VMEM((qpw * _LANES,), f32),
            pltpu.VMEM((qpw * _LANES,), f32),
            pltpu.VMEM((qpw * _LANES,), f32),
            pltpu.VMEM((_CHUNK,), f32),
            pltpu.VMEM((_CHUNK,), f32),
            pltpu.VMEM((_CHUNK,), f32),
            pltpu.VMEM((ncand,), f32),
            pltpu.VMEM((ncand // 128, 128), i32),
            pltpu.VMEM((ncand,), f32),
            pltpu.VMEM((ncand,), f32),
            pltpu.VMEM((ncand,), f32),
            pltpu.SemaphoreType.DMA,
        ],
    )
    return kfn(qx, qy, qz, ox, oy, oz, obs_lat, obs_lon, obs_tgt)


def _refine_body(qlat_ref, qlon_ref, klat_ref, klon_ref, ktgt_ref, kidx_ref,
                 d_ref, i_ref, t_ref):
    ql = qlat_ref[...]
    qn = qlon_ref[...]
    kl = klat_ref[...]
    kn = klon_ref[...]
    kt = ktgt_ref[...]
    ki = kidx_ref[...]

    sdlat = _sinp((ql - kl) * 0.5)
    sdlon = _sinp((qn - kn) * 0.5)
    a = sdlat * sdlat + _cosp(ql) * _cosp(kl) * sdlon * sdlon
    a = jnp.clip(a, 1e-12, 1.0)
    d_all = (2.0 * _RADIUS) * _asinp(jnp.sqrt(a))

    big = jnp.int32(2 ** 30)
    col = lax.broadcasted_iota(jnp.int32, a.shape, 1)
    vals = a
    out_d = jnp.zeros(a.shape, jnp.float32)
    out_i = jnp.zeros(a.shape, jnp.int32)
    out_t = jnp.zeros(a.shape, jnp.float32)
    for j in range(_KNN):
        mn = jnp.min(vals, axis=1, keepdims=True)
        ismn = vals == mn
        pstar = jnp.min(jnp.where(ismn, ki, big), axis=1, keepdims=True)
        pick = ismn & (ki == pstar)
        dj = jnp.sum(jnp.where(pick, d_all, 0.0), axis=1, keepdims=True)
        tj = jnp.sum(jnp.where(pick, kt, 0.0), axis=1, keepdims=True)
        out_d = jnp.where(col == j, dj, out_d)
        out_i = jnp.where(col == j, pstar, out_i)
        out_t = jnp.where(col == j, tj, out_t)
        vals = jnp.where(pick, jnp.inf, vals)

    d_ref[...] = out_d[:, :_KNN]
    i_ref[...] = out_i[:, :_KNN]
    t_ref[...] = out_t[:, :_KNN]


def _refine(qlat, qlon, clat, clon, ctgt, cidx):
    q = qlat.shape[0]
    return pl.pallas_call(
        _refine_body,
        out_shape=[
            jax.ShapeDtypeStruct((q, _KNN), jnp.float32),
            jax.ShapeDtypeStruct((q, _KNN), jnp.int32),
            jax.ShapeDtypeStruct((q, _KNN), jnp.float32),
        ],
    )(qlat, qlon, clat, clon, ctgt, cidx)


def kernel(query_lat, query_lon, obs_lat, obs_lon, obs_targets):
    q = query_lat.shape[0]
    k = obs_lat.shape[0]
    kp = ((k + 127) // 128) * 128

    olat2 = jnp.pad(obs_lat, (0, kp - k)).reshape(kp // 128, 128)
    olon2 = jnp.pad(obs_lon, (0, kp - k)).reshape(kp // 128, 128)
    ox2, oy2, oz2 = _features(olat2, olon2)
    ox = ox2.reshape(kp)[:k]
    oy = oy2.reshape(kp)[:k]
    oz = oz2.reshape(kp)[:k]

    qx2, qy2, qz2 = _features(query_lat.reshape(q // 128, 128),
                              query_lon.reshape(q // 128, 128))
    qx = qx2.reshape(q)
    qy = qy2.reshape(q)
    qz = qz2.reshape(q)

    oi, clat, clon, ctgt = _topk_sc(
        jnp.repeat(qx, _LANES), jnp.repeat(qy, _LANES),
        jnp.repeat(qz, _LANES), ox, oy, oz,
        obs_lat, obs_lon, obs_targets)

    dists, indices, targets = _refine(
        query_lat.reshape(q, 1), query_lon.reshape(q, 1),
        clat.reshape(q, _M), clon.reshape(q, _M), ctgt.reshape(q, _M),
        oi.reshape(q, _M))
    return dists, indices, targets
